# Initial kernel scaffold; baseline (speedup 1.0000x reference)
import jax
import jax.numpy as jnp
from jax.experimental import pallas as pl


def kernel(embeddings, embed):
    raise NotImplementedError("write your pallas kernel here")


# speedup vs baseline: 1.1921x; 1.1921x over previous
import functools

import jax
import jax.numpy as jnp
from jax.experimental import pallas as pl


def _rvq_body(x_ref, emb_ref, embt_ref, cbn_ref, out_ref, codes_ref, *, Q, K, TB):
    r0 = x_ref[0]
    r = r0
    qt = jnp.zeros_like(r0)
    rows = []
    kio = jax.lax.broadcasted_iota(jnp.int32, (K, TB), 0)
    for q in range(Q):
        cb = emb_ref[q]
        cbt = embt_ref[q]
        rn = jnp.sum(r * r, axis=0, keepdims=True)
        mm = jnp.dot(cb, r)
        dist = (rn - 2.0 * mm) + cbn_ref[q]
        m = jnp.min(dist, axis=0, keepdims=True)
        idx = jnp.min(jnp.where(dist == m, kio, K), axis=0, keepdims=True)
        oh = (kio == idx).astype(jnp.float32)
        quant = jnp.dot(cbt, oh, precision=jax.lax.Precision.HIGHEST)
        r = r - quant
        qt = qt + quant
        rows.append(idx)
    codes_ref[0] = jnp.concatenate(rows, axis=0)
    out_ref[0] = qt


def kernel(embeddings, embed):
    B, D, T = embeddings.shape
    Q, K, _ = embed.shape
    TB = min(512, T)
    embed_t = jnp.transpose(embed, (0, 2, 1))
    cbn = jnp.sum(embed * embed, axis=-1)[..., None]

    body = functools.partial(_rvq_body, Q=Q, K=K, TB=TB)
    quant, codes_t = pl.pallas_call(
        body,
        grid=(B, T // TB),
        in_specs=[
            pl.BlockSpec((1, D, TB), lambda b, t: (b, 0, t)),
            pl.BlockSpec((Q, K, D), lambda b, t: (0, 0, 0)),
            pl.BlockSpec((Q, D, K), lambda b, t: (0, 0, 0)),
            pl.BlockSpec((Q, K, 1), lambda b, t: (0, 0, 0)),
        ],
        out_specs=[
            pl.BlockSpec((1, D, TB), lambda b, t: (b, 0, t)),
            pl.BlockSpec((1, Q, TB), lambda b, t: (b, 0, t)),
        ],
        out_shape=[
            jax.ShapeDtypeStruct((B, D, T), jnp.float32),
            jax.ShapeDtypeStruct((B, Q, T), jnp.int32),
        ],
    )(embeddings, embed, embed_t, cbn)
    return quant, jnp.transpose(codes_t, (1, 0, 2))


# speedup vs baseline: 1.8692x; 1.5679x over previous
import functools

import jax
import jax.numpy as jnp
from jax.experimental import pallas as pl


def _rvq_body(x_ref, emb_ref, c1_ref, c2_ref, c3_ref, cbn_ref,
              out_ref, codes_ref, *, Q, K, TB):
    r0 = x_ref[0]
    r = r0
    qt = jnp.zeros_like(r0)
    rows = []
    kio = jax.lax.broadcasted_iota(jnp.int32, (K, TB), 0)
    f32 = jnp.float32
    for q in range(Q):
        cb = emb_ref[q]
        rn = jnp.sum(r * r, axis=0, keepdims=True)
        mm = jnp.dot(cb, r)
        dist = (rn - 2.0 * mm) + cbn_ref[q]
        m = jnp.min(dist, axis=0, keepdims=True)
        idx = jnp.min(jnp.where(dist == m, kio, K), axis=0, keepdims=True)
        oh = (kio == idx).astype(jnp.bfloat16)
        quant = (jnp.dot(c1_ref[q], oh, preferred_element_type=f32).astype(f32)
                 + jnp.dot(c2_ref[q], oh, preferred_element_type=f32).astype(f32)
                 + jnp.dot(c3_ref[q], oh, preferred_element_type=f32).astype(f32))
        r = r - quant
        qt = qt + quant
        rows.append(idx)
    codes_ref[0] = jnp.concatenate(rows, axis=0)
    out_ref[0] = qt


def kernel(embeddings, embed):
    B, D, T = embeddings.shape
    Q, K, _ = embed.shape
    TB = min(512, T)
    embed_t = jnp.transpose(embed, (0, 2, 1))

    def _trunc16(x):
        xi = jax.lax.bitcast_convert_type(x, jnp.uint32)
        return jax.lax.bitcast_convert_type(xi & jnp.uint32(0xFFFF0000), jnp.float32)

    c1f = _trunc16(embed_t)
    r1 = embed_t - c1f
    c2f = _trunc16(r1)
    r2 = r1 - c2f
    c1 = c1f.astype(jnp.bfloat16)
    c2 = c2f.astype(jnp.bfloat16)
    c3 = r2.astype(jnp.bfloat16)
    cbn = jnp.sum(embed * embed, axis=-1)[..., None]

    body = functools.partial(_rvq_body, Q=Q, K=K, TB=TB)
    quant, codes_t = pl.pallas_call(
        body,
        grid=(B, T // TB),
        in_specs=[
            pl.BlockSpec((1, D, TB), lambda b, t: (b, 0, t)),
            pl.BlockSpec((Q, K, D), lambda b, t: (0, 0, 0)),
            pl.BlockSpec((Q, D, K), lambda b, t: (0, 0, 0)),
            pl.BlockSpec((Q, D, K), lambda b, t: (0, 0, 0)),
            pl.BlockSpec((Q, D, K), lambda b, t: (0, 0, 0)),
            pl.BlockSpec((Q, K, 1), lambda b, t: (0, 0, 0)),
        ],
        out_specs=[
            pl.BlockSpec((1, D, TB), lambda b, t: (b, 0, t)),
            pl.BlockSpec((1, Q, TB), lambda b, t: (b, 0, t)),
        ],
        out_shape=[
            jax.ShapeDtypeStruct((B, D, T), jnp.float32),
            jax.ShapeDtypeStruct((B, Q, T), jnp.int32),
        ],
    )(embeddings, embed, c1, c2, c3, cbn)
    return quant, jnp.transpose(codes_t, (1, 0, 2))


# speedup vs baseline: 2.2253x; 1.1905x over previous
import functools

import jax
import jax.numpy as jnp
from jax.experimental import pallas as pl


def _rvq_body(x_ref, emb_ref, c1_ref, c2_ref, c3_ref, cbn_ref,
              out_ref, codes_ref, *, Q, K, TB):
    r0 = x_ref[0]
    r = r0
    qt = jnp.zeros_like(r0)
    rows = []
    kio = jax.lax.broadcasted_iota(jnp.int32, (K, TB), 0)
    f32 = jnp.float32
    for q in range(Q):
        cb = emb_ref[q]
        rn = jnp.sum(r * r, axis=0, keepdims=True)
        mm = jnp.dot(cb, r)
        dist = (rn - 2.0 * mm) + cbn_ref[q]
        m = jnp.min(dist, axis=0, keepdims=True)
        idx = jnp.min(jnp.where(dist == m, kio, K), axis=0, keepdims=True)
        oh = (kio == idx).astype(jnp.bfloat16)
        quant = (jnp.dot(c1_ref[q], oh, preferred_element_type=f32).astype(f32)
                 + jnp.dot(c2_ref[q], oh, preferred_element_type=f32).astype(f32)
                 + jnp.dot(c3_ref[q], oh, preferred_element_type=f32).astype(f32))
        r = r - quant
        qt = qt + quant
        rows.append(idx)
    codes_ref[0] = jnp.concatenate(rows, axis=0)
    out_ref[0] = qt


def kernel(embeddings, embed):
    B, D, T = embeddings.shape
    Q, K, _ = embed.shape
    TB = min(1024, T)
    embed_t = jnp.transpose(embed, (0, 2, 1))

    def _trunc16(x):
        xi = jax.lax.bitcast_convert_type(x, jnp.uint32)
        return jax.lax.bitcast_convert_type(xi & jnp.uint32(0xFFFF0000), jnp.float32)

    c1f = _trunc16(embed_t)
    r1 = embed_t - c1f
    c2f = _trunc16(r1)
    r2 = r1 - c2f
    c1 = c1f.astype(jnp.bfloat16)
    c2 = c2f.astype(jnp.bfloat16)
    c3 = r2.astype(jnp.bfloat16)
    cbn = jnp.sum(embed * embed, axis=-1)[..., None]

    body = functools.partial(_rvq_body, Q=Q, K=K, TB=TB)
    quant, codes_t = pl.pallas_call(
        body,
        grid=(B, T // TB),
        in_specs=[
            pl.BlockSpec((1, D, TB), lambda b, t: (b, 0, t)),
            pl.BlockSpec((Q, K, D), lambda b, t: (0, 0, 0)),
            pl.BlockSpec((Q, D, K), lambda b, t: (0, 0, 0)),
            pl.BlockSpec((Q, D, K), lambda b, t: (0, 0, 0)),
            pl.BlockSpec((Q, D, K), lambda b, t: (0, 0, 0)),
            pl.BlockSpec((Q, K, 1), lambda b, t: (0, 0, 0)),
        ],
        out_specs=[
            pl.BlockSpec((1, D, TB), lambda b, t: (b, 0, t)),
            pl.BlockSpec((1, Q, TB), lambda b, t: (b, 0, t)),
        ],
        out_shape=[
            jax.ShapeDtypeStruct((B, D, T), jnp.float32),
            jax.ShapeDtypeStruct((B, Q, T), jnp.int32),
        ],
    )(embeddings, embed, c1, c2, c3, cbn)
    return quant, jnp.transpose(codes_t, (1, 0, 2))
